# initial kernel scaffold (unmeasured)
import jax
import jax.numpy as jnp
from jax import lax
from jax.experimental import pallas as pl
from jax.experimental.pallas import tpu as pltpu

N_DEV = 4
SQ = 1024
D = 1024
HG = 8
DH = 128
BLK = 64
SCALE = 0.08838834764831843
F32 = jnp.float32
BF16 = jnp.bfloat16

_CompilerParams = getattr(pltpu, "CompilerParams", None) or getattr(
    pltpu, "TPUCompilerParams"
)


def kernel(x, Wq, K_ext, V_ext, Wo):
    def body(x_ref, wq_ref, k_hbm, v_hbm, wo_ref, out_ref,
             comm_ref, xbf_ref, ctx_ref, kv_ref,
             kv_sems, send_sems, recv_sems):
        my = lax.axis_index("i")
        left = lax.rem(my + N_DEV - 1, N_DEV)
        right = lax.rem(my + 1, N_DEV)

        barrier = pltpu.get_barrier_semaphore()
        for nbr in (left, right):
            pl.semaphore_signal(barrier, inc=1, device_id=(nbr,),
                                device_id_type=pl.DeviceIdType.MESH)
        pl.semaphore_wait(barrier, 2)

        xbf_ref[...] = x_ref[0].astype(BF16)
        comm_ref[0, 0] = wq_ref[...].astype(BF16)
        comm_ref[0, 1] = wo_ref[...].astype(BF16)

        qb = lax.broadcasted_iota(jnp.int32, (SQ, SQ), 0) // BLK
        kb = lax.broadcasted_iota(jnp.int32, (SQ, SQ), 1) // BLK
        mask = kb <= qb

        def start_kv(s):
            g = lax.rem(my - s + N_DEV, N_DEV)
            kc = pltpu.make_async_copy(
                k_hbm.at[my, :, pl.ds(g * HG, HG), :], kv_ref.at[0],
                kv_sems.at[0])
            vc = pltpu.make_async_copy(
                v_hbm.at[my, :, pl.ds(g * HG, HG), :], kv_ref.at[1],
                kv_sems.at[1])
            kc.start()
            vc.start()
            return kc, vc

        for s in range(N_DEV):
            if s < N_DEV - 1:
                rdma = pltpu.make_async_remote_copy(
                    src_ref=comm_ref.at[s],
                    dst_ref=comm_ref.at[s + 1],
                    send_sem=send_sems.at[s],
                    recv_sem=recv_sems.at[s],
                    device_id=(right,),
                    device_id_type=pl.DeviceIdType.MESH,
                )
                rdma.start()
            kc, vc = start_kv(s)
            wq_g = comm_ref[s, 0]
            wo_g = comm_ref[s, 1]
            q = lax.dot(xbf_ref[...], wq_g,
                        preferred_element_type=F32).astype(BF16)
            kc.wait()
            vc.wait()
            for h in range(HG):
                qh = q[:, h * DH:(h + 1) * DH]
                kh = kv_ref[0, :, h, :].astype(BF16)
                sc = lax.dot_general(
                    qh, kh, (((1,), (1,)), ((), ())),
                    preferred_element_type=F32) * SCALE
                sc = jnp.where(mask, sc, -1e9)
                m = jnp.max(sc, axis=1, keepdims=True)
                w = jnp.exp(sc - m)
                denom = jnp.sum(w, axis=1, keepdims=True)
                wbf = (w / denom).astype(BF16)
                vh = kv_ref[1, :, h, :].astype(BF16)
                ctx_ref[:, h * DH:(h + 1) * DH] = lax.dot(
                    wbf, vh, preferred_element_type=F32).astype(BF16)
            contrib = lax.dot(ctx_ref[...], wo_g,
                              preferred_element_type=F32)
            if s == 0:
                out_ref[0] = contrib
            else:
                out_ref[0] += contrib
            if s < N_DEV - 1:
                rdma.wait()

    return pl.pallas_call(
        body,
        out_shape=jax.ShapeDtypeStruct((1, SQ, D), jnp.float32),
        in_specs=[
            pl.BlockSpec(memory_space=pltpu.VMEM),
            pl.BlockSpec(memory_space=pltpu.VMEM),
            pl.BlockSpec(memory_space=pltpu.ANY),
            pl.BlockSpec(memory_space=pltpu.ANY),
            pl.BlockSpec(memory_space=pltpu.VMEM),
        ],
        out_specs=pl.BlockSpec(memory_space=pltpu.VMEM),
        scratch_shapes=[
            pltpu.VMEM((N_DEV, 2, D, D), BF16),
            pltpu.VMEM((SQ, D), BF16),
            pltpu.VMEM((SQ, D), BF16),
            pltpu.VMEM((2, SQ, HG, DH), F32),
            pltpu.SemaphoreType.DMA((2,)),
            pltpu.SemaphoreType.DMA((N_DEV - 1,)),
            pltpu.SemaphoreType.DMA((N_DEV - 1,)),
        ],
        compiler_params=_CompilerParams(collective_id=0),
    )(x, Wq, K_ext, V_ext, Wo)


# baseline (device time: 183090 ns/iter reference)
import jax
import jax.numpy as jnp
from jax import lax
from jax.experimental import pallas as pl
from jax.experimental.pallas import tpu as pltpu

N_DEV = 4
SQ = 1024
D = 1024
HG = 8
DH = 128
BLK = 64
SCALE = 0.08838834764831843
F32 = jnp.float32
BF16 = jnp.bfloat16

_CompilerParams = getattr(pltpu, "CompilerParams", None) or getattr(
    pltpu, "TPUCompilerParams"
)


def kernel(x, Wq, K_ext, V_ext, Wo):
    def body(x_ref, wq_ref, k_hbm, v_hbm, wo_ref, out_ref,
             comm_ref, xbf_ref, ctx_ref, kv_ref,
             kv_sems, send_sems, recv_sems):
        my = lax.axis_index("i")
        left = lax.rem(my + N_DEV - 1, N_DEV)
        right = lax.rem(my + 1, N_DEV)

        barrier = pltpu.get_barrier_semaphore()
        for nbr in (left, right):
            pl.semaphore_signal(barrier, inc=1, device_id=(nbr,),
                                device_id_type=pl.DeviceIdType.MESH)
        pl.semaphore_wait(barrier, 2)

        xbf_ref[...] = x_ref[0].astype(BF16)
        comm_ref[0, 0] = wq_ref[...].astype(BF16)
        comm_ref[0, 1] = wo_ref[...].astype(BF16)

        qb = lax.broadcasted_iota(jnp.int32, (SQ, SQ), 0) // BLK
        kb = lax.broadcasted_iota(jnp.int32, (SQ, SQ), 1) // BLK
        mask = kb <= qb

        def start_kv(s):
            g = lax.rem(my - s + N_DEV, N_DEV)
            kc = pltpu.make_async_copy(
                k_hbm.at[my, :, pl.ds(g * HG, HG), :], kv_ref.at[0],
                kv_sems.at[0])
            vc = pltpu.make_async_copy(
                v_hbm.at[my, :, pl.ds(g * HG, HG), :], kv_ref.at[1],
                kv_sems.at[1])
            kc.start()
            vc.start()
            return kc, vc

        for s in range(N_DEV):
            if s < N_DEV - 1:
                rdma = pltpu.make_async_remote_copy(
                    src_ref=comm_ref.at[s],
                    dst_ref=comm_ref.at[s + 1],
                    send_sem=send_sems.at[s],
                    recv_sem=recv_sems.at[s],
                    device_id=(right,),
                    device_id_type=pl.DeviceIdType.MESH,
                )
                rdma.start()
            kc, vc = start_kv(s)
            wq_g = comm_ref[s, 0]
            wo_g = comm_ref[s, 1]
            q = lax.dot(xbf_ref[...], wq_g,
                        preferred_element_type=F32).astype(BF16)
            kc.wait()
            vc.wait()
            for h in range(HG):
                qh = q[:, h * DH:(h + 1) * DH]
                kh = kv_ref[0, :, h, :].astype(BF16)
                sc = lax.dot_general(
                    qh, kh, (((1,), (1,)), ((), ())),
                    preferred_element_type=F32) * SCALE
                sc = jnp.where(mask, sc, -1e9)
                m = jnp.max(sc, axis=1, keepdims=True)
                w = jnp.exp(sc - m)
                denom = jnp.sum(w, axis=1, keepdims=True)
                wbf = (w / denom).astype(BF16)
                vh = kv_ref[1, :, h, :].astype(BF16)
                ctx_ref[:, h * DH:(h + 1) * DH] = lax.dot(
                    wbf, vh, preferred_element_type=F32).astype(BF16)
            contrib = lax.dot(ctx_ref[...], wo_g,
                              preferred_element_type=F32)
            if s == 0:
                out_ref[0] = contrib
            else:
                out_ref[0] += contrib
            if s < N_DEV - 1:
                rdma.wait()

    return pl.pallas_call(
        body,
        out_shape=jax.ShapeDtypeStruct((1, SQ, D), jnp.float32),
        in_specs=[
            pl.BlockSpec(memory_space=pltpu.MemorySpace.VMEM),
            pl.BlockSpec(memory_space=pltpu.MemorySpace.VMEM),
            pl.BlockSpec(memory_space=pl.ANY),
            pl.BlockSpec(memory_space=pl.ANY),
            pl.BlockSpec(memory_space=pltpu.MemorySpace.VMEM),
        ],
        out_specs=pl.BlockSpec(memory_space=pltpu.MemorySpace.VMEM),
        scratch_shapes=[
            pltpu.VMEM((N_DEV, 2, D, D), BF16),
            pltpu.VMEM((SQ, D), BF16),
            pltpu.VMEM((SQ, D), BF16),
            pltpu.VMEM((2, SQ, HG, DH), F32),
            pltpu.SemaphoreType.DMA((2,)),
            pltpu.SemaphoreType.DMA((N_DEV - 1,)),
            pltpu.SemaphoreType.DMA((N_DEV - 1,)),
        ],
        compiler_params=_CompilerParams(
            collective_id=0, vmem_limit_bytes=100 * 1024 * 1024),
    )(x, Wq, K_ext, V_ext, Wo)


# device time: 114248 ns/iter; 1.6026x vs baseline; 1.6026x over previous
import jax
import jax.numpy as jnp
from jax import lax
from jax.experimental import pallas as pl
from jax.experimental.pallas import tpu as pltpu

N_DEV = 4
SQ = 1024
D = 1024
HG = 8
HH = 4
DH = 128
DHALF = HH * DH
BLK = 64
SCALE = 0.08838834764831843
F32 = jnp.float32
BF16 = jnp.bfloat16

_CompilerParams = getattr(pltpu, "CompilerParams", None) or getattr(
    pltpu, "TPUCompilerParams"
)


def kernel(x, Wq, K_ext, V_ext, Wo):
    def body(x_ref, wq_ref, k_hbm, v_hbm, wo_ref, out_ref,
             cwq_ref, cwo_ref, ccq_ref, cco_ref,
             xbf_ref, ctx_ref, kv_ref,
             kv_sems, send_sems, recv_sems):
        my = lax.axis_index("i")
        left = lax.rem(my + N_DEV - 1, N_DEV)
        right = lax.rem(my + 1, N_DEV)

        barrier = pltpu.get_barrier_semaphore()
        for nbr in (left, right):
            pl.semaphore_signal(barrier, inc=1, device_id=(nbr,),
                                device_id_type=pl.DeviceIdType.MESH)
        pl.semaphore_wait(barrier, 2)

        xbf_ref[...] = x_ref[0].astype(BF16)
        cwq_ref[0] = wq_ref[:, :DHALF].astype(BF16)
        ccq_ref[0] = wq_ref[:, DHALF:].astype(BF16)
        cwo_ref[0] = wo_ref[:DHALF, :].astype(BF16)
        cco_ref[0] = wo_ref[DHALF:, :].astype(BF16)

        qb = lax.broadcasted_iota(jnp.int32, (SQ, SQ), 0) // BLK
        kb = lax.broadcasted_iota(jnp.int32, (SQ, SQ), 1) // BLK
        mask = kb <= qb

        def start_kv(s):
            p = s % 2
            ga = lax.rem(my - s + N_DEV, N_DEV)
            gb = lax.rem(my + s, N_DEV)
            copies = []
            for i, h0 in ((0, ga * HG), (1, ga * HG),
                          (2, gb * HG + HH), (3, gb * HG + HH)):
                src = (k_hbm if i % 2 == 0 else v_hbm)
                copies.append(pltpu.make_async_copy(
                    src.at[my, :, pl.ds(h0, HH), :], kv_ref.at[p, i],
                    kv_sems.at[p, i]))
            for c in copies:
                c.start()
            return copies

        def compute_half(wq_h, wo_h, p, kv_k, kv_v):
            q = lax.dot(xbf_ref[...], wq_h,
                        preferred_element_type=F32).astype(BF16)
            for h in range(HH):
                qh = q[:, h * DH:(h + 1) * DH]
                kh = kv_ref[p, kv_k, :, h, :].astype(BF16)
                sc = lax.dot_general(
                    qh, kh, (((1,), (1,)), ((), ())),
                    preferred_element_type=F32) * SCALE
                sc = jnp.where(mask, sc, -1e9)
                m = jnp.max(sc, axis=1, keepdims=True)
                w = jnp.exp(sc - m)
                denom = jnp.sum(w, axis=1, keepdims=True)
                wbf = (w / denom).astype(BF16)
                vh = kv_ref[p, kv_v, :, h, :].astype(BF16)
                ctx_ref[:, h * DH:(h + 1) * DH] = lax.dot(
                    wbf, vh, preferred_element_type=F32).astype(BF16)
            return lax.dot(ctx_ref[...], wo_h, preferred_element_type=F32)

        kv_copies = start_kv(0)
        for s in range(N_DEV):
            rdmas = []
            if s < N_DEV - 1:
                for i, (buf, dst) in enumerate((
                        (cwq_ref, right), (cwo_ref, right),
                        (ccq_ref, left), (cco_ref, left))):
                    rdma = pltpu.make_async_remote_copy(
                        src_ref=buf.at[s],
                        dst_ref=buf.at[s + 1],
                        send_sem=send_sems.at[i, s],
                        recv_sem=recv_sems.at[i, s],
                        device_id=(dst,),
                        device_id_type=pl.DeviceIdType.MESH,
                    )
                    rdma.start()
                    rdmas.append(rdma)
            for c in kv_copies:
                c.wait()
            if s < N_DEV - 1:
                kv_copies = start_kv(s + 1)
            p = s % 2
            contrib_a = compute_half(cwq_ref[s], cwo_ref[s], p, 0, 1)
            if s == 0:
                out_ref[0] = contrib_a
            else:
                out_ref[0] += contrib_a
            contrib_b = compute_half(ccq_ref[s], cco_ref[s], p, 2, 3)
            out_ref[0] += contrib_b
            if s < N_DEV - 1:
                for r in rdmas:
                    r.wait()

    return pl.pallas_call(
        body,
        out_shape=jax.ShapeDtypeStruct((1, SQ, D), jnp.float32),
        in_specs=[
            pl.BlockSpec(memory_space=pltpu.MemorySpace.VMEM),
            pl.BlockSpec(memory_space=pltpu.MemorySpace.VMEM),
            pl.BlockSpec(memory_space=pl.ANY),
            pl.BlockSpec(memory_space=pl.ANY),
            pl.BlockSpec(memory_space=pltpu.MemorySpace.VMEM),
        ],
        out_specs=pl.BlockSpec(memory_space=pltpu.MemorySpace.VMEM),
        scratch_shapes=[
            pltpu.VMEM((N_DEV, D, DHALF), BF16),
            pltpu.VMEM((N_DEV, DHALF, D), BF16),
            pltpu.VMEM((N_DEV, D, DHALF), BF16),
            pltpu.VMEM((N_DEV, DHALF, D), BF16),
            pltpu.VMEM((SQ, D), BF16),
            pltpu.VMEM((SQ, DHALF), BF16),
            pltpu.VMEM((2, 4, SQ, HH, DH), F32),
            pltpu.SemaphoreType.DMA((2, 4)),
            pltpu.SemaphoreType.DMA((4, N_DEV - 1)),
            pltpu.SemaphoreType.DMA((4, N_DEV - 1)),
        ],
        compiler_params=_CompilerParams(
            collective_id=0, vmem_limit_bytes=100 * 1024 * 1024),
    )(x, Wq, K_ext, V_ext, Wo)


# device time: 106520 ns/iter; 1.7188x vs baseline; 1.0725x over previous
import jax
import jax.numpy as jnp
from jax import lax
from jax.experimental import pallas as pl
from jax.experimental.pallas import tpu as pltpu

N_DEV = 4
SQ = 1024
SQ2 = 512
D = 1024
HG = 8
HH = 4
DH = 128
DHALF = HH * DH
BLK = 64
SCALE = 0.08838834764831843
F32 = jnp.float32
BF16 = jnp.bfloat16

_CompilerParams = getattr(pltpu, "CompilerParams", None) or getattr(
    pltpu, "TPUCompilerParams"
)


def kernel(x, Wq, K_ext, V_ext, Wo):
    def body(x_ref, wq_ref, k_hbm, v_hbm, wo_ref, out_ref,
             cwq_ref, cwo_ref, ccq_ref, cco_ref,
             xbf_ref, ctx_ref, kv_ref,
             kv_sems, send_sems, recv_sems):
        my = lax.axis_index("i")
        left = lax.rem(my + N_DEV - 1, N_DEV)
        right = lax.rem(my + 1, N_DEV)

        barrier = pltpu.get_barrier_semaphore()
        for nbr in (left, right):
            pl.semaphore_signal(barrier, inc=1, device_id=(nbr,),
                                device_id_type=pl.DeviceIdType.MESH)
        pl.semaphore_wait(barrier, 2)

        xbf_ref[...] = x_ref[0].astype(BF16)
        cwq_ref[0] = wq_ref[:, :DHALF].astype(BF16)
        ccq_ref[0] = wq_ref[:, DHALF:].astype(BF16)
        cwo_ref[0] = wo_ref[:DHALF, :].astype(BF16)
        cco_ref[0] = wo_ref[DHALF:, :].astype(BF16)

        qb = lax.broadcasted_iota(jnp.int32, (SQ2, SQ2), 0) // BLK
        kb = lax.broadcasted_iota(jnp.int32, (SQ2, SQ2), 1) // BLK
        mask = kb <= qb

        def start_kv(s):
            p = s % 2
            ga = lax.rem(my - s + N_DEV, N_DEV)
            gb = lax.rem(my + s, N_DEV)
            copies = []
            for i, h0 in ((0, ga * HG), (1, ga * HG),
                          (2, gb * HG + HH), (3, gb * HG + HH)):
                src = (k_hbm if i % 2 == 0 else v_hbm)
                copies.append(pltpu.make_async_copy(
                    src.at[my, :, pl.ds(h0, HH), :], kv_ref.at[p, i],
                    kv_sems.at[p, i]))
            for c in copies:
                c.start()
            return copies

        def compute_half(wq_h, wo_h, p, kv_k, kv_v):
            q = (lax.dot(xbf_ref[...], wq_h, preferred_element_type=F32)
                 * SCALE).astype(BF16)
            dot_t = lambda a, b: lax.dot_general(
                a, b, (((1,), (1,)), ((), ())), preferred_element_type=F32)
            for h in range(HH):
                qh_u = q[:SQ2, h * DH:(h + 1) * DH]
                qh_l = q[SQ2:, h * DH:(h + 1) * DH]
                kh = kv_ref[p, kv_k, :, h, :].astype(BF16)
                vh = kv_ref[p, kv_v, :, h, :].astype(BF16)
                w_u = jnp.where(mask, jnp.exp(dot_t(qh_u, kh[:SQ2])), 0.0)
                den_u = jnp.sum(w_u, axis=1, keepdims=True)
                ctx_u = lax.dot(w_u.astype(BF16), vh[:SQ2],
                                preferred_element_type=F32) / den_u
                s_l = dot_t(qh_l, kh)
                w_ll = jnp.exp(s_l[:, :SQ2])
                w_lr = jnp.where(mask, jnp.exp(s_l[:, SQ2:]), 0.0)
                den_l = (jnp.sum(w_ll, axis=1, keepdims=True)
                         + jnp.sum(w_lr, axis=1, keepdims=True))
                ctx_l = (lax.dot(w_ll.astype(BF16), vh[:SQ2],
                                 preferred_element_type=F32)
                         + lax.dot(w_lr.astype(BF16), vh[SQ2:],
                                   preferred_element_type=F32)) / den_l
                ctx_ref[:SQ2, h * DH:(h + 1) * DH] = ctx_u.astype(BF16)
                ctx_ref[SQ2:, h * DH:(h + 1) * DH] = ctx_l.astype(BF16)
            return lax.dot(ctx_ref[...], wo_h, preferred_element_type=F32)

        kv_copies = start_kv(0)
        for s in range(N_DEV):
            rdmas = []
            if s < N_DEV - 1:
                for i, (buf, dst) in enumerate((
                        (cwq_ref, right), (cwo_ref, right),
                        (ccq_ref, left), (cco_ref, left))):
                    rdma = pltpu.make_async_remote_copy(
                        src_ref=buf.at[s],
                        dst_ref=buf.at[s + 1],
                        send_sem=send_sems.at[i, s],
                        recv_sem=recv_sems.at[i, s],
                        device_id=(dst,),
                        device_id_type=pl.DeviceIdType.MESH,
                    )
                    rdma.start()
                    rdmas.append(rdma)
            for c in kv_copies:
                c.wait()
            if s < N_DEV - 1:
                kv_copies = start_kv(s + 1)
            p = s % 2
            contrib_a = compute_half(cwq_ref[s], cwo_ref[s], p, 0, 1)
            if s == 0:
                out_ref[0] = contrib_a
            else:
                out_ref[0] += contrib_a
            contrib_b = compute_half(ccq_ref[s], cco_ref[s], p, 2, 3)
            out_ref[0] += contrib_b
            if s < N_DEV - 1:
                for r in rdmas:
                    r.wait()

    return pl.pallas_call(
        body,
        out_shape=jax.ShapeDtypeStruct((1, SQ, D), jnp.float32),
        in_specs=[
            pl.BlockSpec(memory_space=pltpu.MemorySpace.VMEM),
            pl.BlockSpec(memory_space=pltpu.MemorySpace.VMEM),
            pl.BlockSpec(memory_space=pl.ANY),
            pl.BlockSpec(memory_space=pl.ANY),
            pl.BlockSpec(memory_space=pltpu.MemorySpace.VMEM),
        ],
        out_specs=pl.BlockSpec(memory_space=pltpu.MemorySpace.VMEM),
        scratch_shapes=[
            pltpu.VMEM((N_DEV, D, DHALF), BF16),
            pltpu.VMEM((N_DEV, DHALF, D), BF16),
            pltpu.VMEM((N_DEV, D, DHALF), BF16),
            pltpu.VMEM((N_DEV, DHALF, D), BF16),
            pltpu.VMEM((SQ, D), BF16),
            pltpu.VMEM((SQ, DHALF), BF16),
            pltpu.VMEM((2, 4, SQ, HH, DH), F32),
            pltpu.SemaphoreType.DMA((2, 4)),
            pltpu.SemaphoreType.DMA((4, N_DEV - 1)),
            pltpu.SemaphoreType.DMA((4, N_DEV - 1)),
        ],
        compiler_params=_CompilerParams(
            collective_id=0, vmem_limit_bytes=100 * 1024 * 1024),
    )(x, Wq, K_ext, V_ext, Wo)


# device time: 102996 ns/iter; 1.7776x vs baseline; 1.0342x over previous
import jax
import jax.numpy as jnp
from jax import lax
from jax.experimental import pallas as pl
from jax.experimental.pallas import tpu as pltpu

N_DEV = 4
SQ = 1024
SQ2 = 512
D = 1024
HG = 8
HH = 4
DH = 128
DHALF = HH * DH
BLK = 64
SCALE = 0.08838834764831843
F32 = jnp.float32
BF16 = jnp.bfloat16

_CompilerParams = getattr(pltpu, "CompilerParams", None) or getattr(
    pltpu, "TPUCompilerParams"
)


def kernel(x, Wq, K_ext, V_ext, Wo):
    def body(x_ref, wq_ref, k_hbm, v_hbm, wo_ref, out_ref,
             qA, oA, qB, oB, xbf_ref, ctx_ref, kv_ref,
             kv_sems, send_sems, recv_sems):
        my = lax.axis_index("i")
        left = lax.rem(my + N_DEV - 1, N_DEV)
        right = lax.rem(my + 1, N_DEV)
        opp = lax.rem(my + 2, N_DEV)

        barrier = pltpu.get_barrier_semaphore()
        for nbr in (left, right):
            pl.semaphore_signal(barrier, inc=1, device_id=(nbr,),
                                device_id_type=pl.DeviceIdType.MESH)
        pl.semaphore_wait(barrier, 2)

        xbf_ref[...] = x_ref[0].astype(BF16)
        qA[my] = wq_ref[:, :DHALF].astype(BF16)
        qB[my] = wq_ref[:, DHALF:].astype(BF16)
        oA[my] = wo_ref[:DHALF, :].astype(BF16)
        oB[my] = wo_ref[DHALF:, :].astype(BF16)

        qb = lax.broadcasted_iota(jnp.int32, (SQ2, SQ2), 0) // BLK
        kb = lax.broadcasted_iota(jnp.int32, (SQ2, SQ2), 1) // BLK
        mask = kb <= qb

        def send_pair(dst, link, j, bufq, bufo, slot):
            out = []
            for t, buf in ((0, bufq), (1, bufo)):
                r = pltpu.make_async_remote_copy(
                    src_ref=buf.at[slot],
                    dst_ref=buf.at[slot],
                    send_sem=send_sems.at[link, j, t],
                    recv_sem=recv_sems.at[link, j, t],
                    device_id=(dst,),
                    device_id_type=pl.DeviceIdType.MESH,
                )
                r.start()
                out.append(r)
            return out

        def wait_pair(link, j, bufq, bufo, slot):
            for t, buf in ((0, bufq), (1, bufo)):
                pltpu.make_async_remote_copy(
                    src_ref=buf.at[slot],
                    dst_ref=buf.at[slot],
                    send_sem=send_sems.at[link, j, t],
                    recv_sem=recv_sems.at[link, j, t],
                    device_id=(left,),
                    device_id_type=pl.DeviceIdType.MESH,
                ).wait_recv()

        kv_plan = [
            (my, 0, my, HH),
            (left, 0, right, HH),
            (left, HH, right, 0),
            (opp, 0, opp, HH),
        ]

        def start_kv(step):
            p = step % 2
            ga, offa, gb, offb = kv_plan[step]
            copies = []
            for i, (g, off) in enumerate(
                    ((ga, offa), (ga, offa), (gb, offb), (gb, offb))):
                src = (k_hbm if i % 2 == 0 else v_hbm)
                copies.append(pltpu.make_async_copy(
                    src.at[my, :, pl.ds(g * HG + off, HH), :],
                    kv_ref.at[p, i], kv_sems.at[p, i]))
            for c in copies:
                c.start()
            return copies

        def compute_half(wq_h, wo_h, p, kv_k, kv_v):
            q = (lax.dot(xbf_ref[...], wq_h, preferred_element_type=F32)
                 * SCALE).astype(BF16)
            dot_t = lambda a, b: lax.dot_general(
                a, b, (((1,), (1,)), ((), ())), preferred_element_type=F32)
            for h in range(HH):
                qh_u = q[:SQ2, h * DH:(h + 1) * DH]
                qh_l = q[SQ2:, h * DH:(h + 1) * DH]
                kh = kv_ref[p, kv_k, :, h, :].astype(BF16)
                vh = kv_ref[p, kv_v, :, h, :].astype(BF16)
                w_u = jnp.where(mask, jnp.exp(dot_t(qh_u, kh[:SQ2])), 0.0)
                den_u = jnp.sum(w_u, axis=1, keepdims=True)
                ctx_u = lax.dot(w_u.astype(BF16), vh[:SQ2],
                                preferred_element_type=F32) / den_u
                s_l = dot_t(qh_l, kh)
                w_ll = jnp.exp(s_l[:, :SQ2])
                w_lr = jnp.where(mask, jnp.exp(s_l[:, SQ2:]), 0.0)
                den_l = (jnp.sum(w_ll, axis=1, keepdims=True)
                         + jnp.sum(w_lr, axis=1, keepdims=True))
                ctx_l = (lax.dot(w_ll.astype(BF16), vh[:SQ2],
                                 preferred_element_type=F32)
                         + lax.dot(w_lr.astype(BF16), vh[SQ2:],
                                   preferred_element_type=F32)) / den_l
                ctx_ref[:SQ2, h * DH:(h + 1) * DH] = ctx_u.astype(BF16)
                ctx_ref[SQ2:, h * DH:(h + 1) * DH] = ctx_l.astype(BF16)
            return lax.dot(ctx_ref[...], wo_h, preferred_element_type=F32)

        sends = []
        sends += send_pair(right, 0, 0, qA, oA, my)
        sends += send_pair(left, 1, 0, qB, oB, my)
        sends += send_pair(right, 0, 1, qB, oB, my)
        sends += send_pair(left, 1, 1, qA, oA, my)
        kv_copies = start_kv(0)
        for c in kv_copies:
            c.wait()
        kv_copies = start_kv(1)
        out_ref[0] = compute_half(qA[my], oA[my], 0, 0, 1)
        out_ref[0] += compute_half(qB[my], oB[my], 0, 2, 3)

        wait_pair(0, 0, qA, oA, left)
        wait_pair(1, 0, qB, oB, right)
        sends += send_pair(right, 0, 2, qA, oA, left)
        sends += send_pair(left, 1, 2, qB, oB, right)
        for c in kv_copies:
            c.wait()
        kv_copies = start_kv(2)
        out_ref[0] += compute_half(qA[left], oA[left], 1, 0, 1)
        out_ref[0] += compute_half(qB[right], oB[right], 1, 2, 3)

        wait_pair(0, 1, qB, oB, left)
        wait_pair(1, 1, qA, oA, right)
        for c in kv_copies:
            c.wait()
        kv_copies = start_kv(3)
        out_ref[0] += compute_half(qB[left], oB[left], 0, 0, 1)
        out_ref[0] += compute_half(qA[right], oA[right], 0, 2, 3)

        wait_pair(0, 2, qA, oA, opp)
        wait_pair(1, 2, qB, oB, opp)
        for c in kv_copies:
            c.wait()
        out_ref[0] += compute_half(qA[opp], oA[opp], 1, 0, 1)
        out_ref[0] += compute_half(qB[opp], oB[opp], 1, 2, 3)

        for r in sends:
            r.wait_send()

    return pl.pallas_call(
        body,
        out_shape=jax.ShapeDtypeStruct((1, SQ, D), jnp.float32),
        in_specs=[
            pl.BlockSpec(memory_space=pltpu.MemorySpace.VMEM),
            pl.BlockSpec(memory_space=pltpu.MemorySpace.VMEM),
            pl.BlockSpec(memory_space=pl.ANY),
            pl.BlockSpec(memory_space=pl.ANY),
            pl.BlockSpec(memory_space=pltpu.MemorySpace.VMEM),
        ],
        out_specs=pl.BlockSpec(memory_space=pltpu.MemorySpace.VMEM),
        scratch_shapes=[
            pltpu.VMEM((N_DEV, D, DHALF), BF16),
            pltpu.VMEM((N_DEV, DHALF, D), BF16),
            pltpu.VMEM((N_DEV, D, DHALF), BF16),
            pltpu.VMEM((N_DEV, DHALF, D), BF16),
            pltpu.VMEM((SQ, D), BF16),
            pltpu.VMEM((SQ, DHALF), BF16),
            pltpu.VMEM((2, 4, SQ, HH, DH), F32),
            pltpu.SemaphoreType.DMA((2, 4)),
            pltpu.SemaphoreType.DMA((2, 3, 2)),
            pltpu.SemaphoreType.DMA((2, 3, 2)),
        ],
        compiler_params=_CompilerParams(
            collective_id=0, vmem_limit_bytes=100 * 1024 * 1024),
    )(x, Wq, K_ext, V_ext, Wo)
